# baseline (speedup 1.0000x reference)
import functools

import jax
import jax.numpy as jnp
from jax import lax
from jax.experimental import pallas as pl
from jax.experimental.pallas import tpu as pltpu
from jax.experimental.pallas import tpu_sc as plsc

B = 4
N = 2048
D = 128
NBINS = 16
BS = 128
K = 16

NROWS = B * N
NC = 2
NS = 16
NW = NC * NS
ROWS_PER_W = NROWS // NW
TILE = 16
ITERS = ROWS_PER_W // TILE


def _fused_body(x_ref, cb_ref, vals_ref, cols_ref, rowf_ref):
    b = pl.program_id(0)
    x = x_ref[0]
    cb = cb_ref[...]

    mul = jnp.dot(x, cb, preferred_element_type=jnp.float32)
    cmul = jnp.concatenate([mul, -mul], axis=1)
    m0 = jnp.max(cmul, axis=1, keepdims=True)
    iota_n16 = lax.broadcasted_iota(jnp.int32, (N, NBINS), 1)
    binc = jnp.min(jnp.where(cmul == m0, iota_n16, NBINS), axis=1,
                   keepdims=True)
    oh16 = (iota_n16 == binc).astype(jnp.float32)

    r = lax.broadcasted_iota(jnp.int32, (BS, BS), 0)
    c = lax.broadcasted_iota(jnp.int32, (BS, BS), 1)
    trilex = (c < r).astype(jnp.float32)

    cumin = []
    bs_rows = []
    for t in range(NBINS):
        blk = oh16[t * BS:(t + 1) * BS, :]
        cumin.append(jnp.dot(trilex, blk, preferred_element_type=jnp.float32))
        bs_rows.append(jnp.sum(blk, axis=0, keepdims=True))
    bsums = jnp.concatenate(bs_rows, axis=0)

    r16 = lax.broadcasted_iota(jnp.int32, (NBINS, NBINS), 0)
    c16 = lax.broadcasted_iota(jnp.int32, (NBINS, NBINS), 1)
    trilex16 = (c16 < r16).astype(jnp.float32)
    striu16 = (r16 < c16).astype(jnp.float32)
    cumblk = jnp.dot(trilex16, bsums, preferred_element_type=jnp.float32)
    counts = jnp.sum(bsums, axis=0, keepdims=True)
    offsets = jnp.dot(counts, striu16, preferred_element_type=jnp.float32)

    dlist = []
    for t in range(NBINS):
        blk = oh16[t * BS:(t + 1) * BS, :]
        pos = cumin[t] + cumblk[t:t + 1, :] + offsets
        dlist.append(jnp.sum(blk * pos, axis=1, keepdims=True))
    dcol = jnp.concatenate(dlist, axis=0).astype(jnp.int32)

    iota_sub = lax.broadcasted_iota(jnp.int32, (N, BS), 0).astype(jnp.float32)
    sim_rows = []
    olane_rows = []
    for cc in range(NBINS):
        svec_row = lax.broadcasted_iota(jnp.int32, (1, BS), 1) + cc * BS
        ohT = dcol == svec_row
        order_lane = jnp.sum(jnp.where(ohT, iota_sub, 0.0), axis=0,
                             keepdims=True)
        olane_rows.append(jnp.broadcast_to(order_lane, (BS, BS)))
        ohTf = ohT.astype(jnp.float32)
        binned = lax.dot_general(ohTf, x, (((0,), (0,)), ((), ())),
                                 preferred_element_type=jnp.float32)
        sim_rows.append(
            lax.dot_general(binned, binned, (((1,), (1,)), ((), ())),
                            preferred_element_type=jnp.float32))

    sim = jnp.concatenate(sim_rows, axis=0)
    olane = jnp.concatenate(olane_rows, axis=0)
    sm = 1.0 / (1.0 + jnp.exp(-sim))

    iota_l = lax.broadcasted_iota(jnp.int32, (N, BS), 1).astype(jnp.float32)
    keymat = iota_l * float(N) + olane

    smod = lax.broadcasted_iota(jnp.int32, (N, 1), 0)
    smod = smod - (smod // BS) * BS
    diag = iota_l == smod.astype(jnp.float32)
    order_col = jnp.sum(jnp.where(diag, olane, 0.0), axis=1, keepdims=True)

    big = float(N * BS * 2)
    work = sm
    vlist = []
    klist = []
    for _ in range(K):
        m = jnp.max(work, axis=1, keepdims=True)
        kmin = jnp.min(jnp.where(work == m, keymat, big), axis=1,
                       keepdims=True)
        vlist.append(m)
        klist.append(kmin)
        work = jnp.where(keymat == kmin, -1.0, work)

    keys = jnp.concatenate(klist, axis=1)
    gcols = keys - jnp.floor(keys * (1.0 / float(N))) * float(N)
    vals_ref[...] = jnp.concatenate(vlist, axis=1)
    cols_ref[...] = gcols.astype(jnp.int32)
    rowf_ref[...] = order_col.astype(jnp.int32) + b * N


_fused_in_specs = [
    pl.BlockSpec((1, N, D), lambda b: (b, 0, 0)),
    pl.BlockSpec((D, NBINS // 2), lambda b: (0, 0)),
]
_fused_out_specs = [
    pl.BlockSpec((N, K), lambda b: (b, 0)),
    pl.BlockSpec((N, K), lambda b: (b, 0)),
    pl.BlockSpec((N, 1), lambda b: (b, 0)),
]
_fused_out_shape = [
    jax.ShapeDtypeStruct((NROWS, K), jnp.float32),
    jax.ShapeDtypeStruct((NROWS, K), jnp.int32),
    jax.ShapeDtypeStruct((NROWS, 1), jnp.int32),
]

_fused = pl.pallas_call(
    _fused_body,
    grid=(B,),
    in_specs=_fused_in_specs,
    out_specs=_fused_out_specs,
    out_shape=_fused_out_shape,
)


@functools.cache
def _make_sc_scatter():
    mesh = plsc.VectorSubcoreMesh(core_axis_name="c", subcore_axis_name="s")

    @functools.partial(
        pl.kernel,
        mesh=mesh,
        compiler_params=pltpu.CompilerParams(needs_layout_passes=False),
        out_type=jax.ShapeDtypeStruct((NROWS, N), jnp.float32),
        scratch_types=[
            pltpu.VMEM((TILE, N), jnp.float32),
            pltpu.VMEM((ROWS_PER_W, K), jnp.float32),
            pltpu.VMEM((ROWS_PER_W, K), jnp.int32),
            pltpu.VMEM((ITERS, TILE), jnp.int32),
        ],
    )
    def _sc_scatter(vals_hbm, cols_hbm, rowf_hbm, zrow_hbm, out_hbm,
                    obuf, vbuf, cbuf, rbuf):
        wid = lax.axis_index("s") * NC + lax.axis_index("c")
        base = wid * ROWS_PER_W
        pltpu.sync_copy(zrow_hbm, obuf)
        pltpu.sync_copy(vals_hbm.at[pl.ds(base, ROWS_PER_W)], vbuf)
        pltpu.sync_copy(cols_hbm.at[pl.ds(base, ROWS_PER_W)], cbuf)
        pltpu.sync_copy(rowf_hbm.at[pl.ds(wid * ITERS, ITERS)], rbuf)
        zeros16 = jnp.zeros((K,), jnp.float32)
        for it in range(ITERS):
            for j in range(TILE):
                rsp = jnp.full((K,), j, jnp.int32)
                plsc.store_scatter(obuf, [rsp, cbuf[it * TILE + j, :]],
                                   vbuf[it * TILE + j, :])
            pltpu.sync_copy(obuf, out_hbm.at[rbuf.at[it]])
            for j in range(TILE):
                rsp = jnp.full((K,), j, jnp.int32)
                plsc.store_scatter(obuf, [rsp, cbuf[it * TILE + j, :]],
                                   zeros16)

    return _sc_scatter


def kernel(x, codebook):
    vals, cols, rowf = _fused(x, codebook)
    rowf2 = rowf.reshape(NROWS // TILE, TILE)
    zrow = jnp.zeros((TILE, N), jnp.float32)
    a = _make_sc_scatter()(vals, cols, rowf2, zrow)
    return a.reshape(B, N, N)


# speedup vs baseline: 1.0430x; 1.0430x over previous
import functools

import jax
import jax.numpy as jnp
from jax import lax
from jax.experimental import pallas as pl
from jax.experimental.pallas import tpu as pltpu
from jax.experimental.pallas import tpu_sc as plsc

B = 4
N = 2048
D = 128
NBINS = 16
BS = 128
K = 16

NROWS = B * N
NC = 2
NS = 16
NW = NC * NS
ROWS_PER_W = NROWS // NW
TILE = 16
ITERS = ROWS_PER_W // TILE


def _hash_sort_body(x_ref, cb_ref, dest_ref):
    x = x_ref[0]
    cb = cb_ref[...]
    mul = jnp.dot(x, cb, preferred_element_type=jnp.float32)
    cmul = jnp.concatenate([mul, -mul], axis=1)
    m = jnp.max(cmul, axis=1, keepdims=True)
    iota_n16 = lax.broadcasted_iota(jnp.int32, (N, NBINS), 1)
    binc = jnp.min(jnp.where(cmul == m, iota_n16, NBINS), axis=1,
                   keepdims=True)
    oh = (iota_n16 == binc).astype(jnp.float32)

    r = lax.broadcasted_iota(jnp.int32, (BS, BS), 0)
    c = lax.broadcasted_iota(jnp.int32, (BS, BS), 1)
    trilex = (c < r).astype(jnp.float32)

    cumin = []
    bs_rows = []
    for t in range(NBINS):
        blk = oh[t * BS:(t + 1) * BS, :]
        cumin.append(jnp.dot(trilex, blk, preferred_element_type=jnp.float32))
        bs_rows.append(jnp.sum(blk, axis=0, keepdims=True))
    bsums = jnp.concatenate(bs_rows, axis=0)

    r16 = lax.broadcasted_iota(jnp.int32, (NBINS, NBINS), 0)
    c16 = lax.broadcasted_iota(jnp.int32, (NBINS, NBINS), 1)
    trilex16 = (c16 < r16).astype(jnp.float32)
    striu16 = (r16 < c16).astype(jnp.float32)
    cumblk = jnp.dot(trilex16, bsums, preferred_element_type=jnp.float32)
    counts = jnp.sum(bsums, axis=0, keepdims=True)
    offsets = jnp.dot(counts, striu16, preferred_element_type=jnp.float32)

    for t in range(NBINS):
        blk = oh[t * BS:(t + 1) * BS, :]
        pos = cumin[t] + cumblk[t:t + 1, :] + offsets
        dest_t = jnp.sum(blk * pos, axis=1, keepdims=True)
        dest_ref[0, t * BS:(t + 1) * BS, :] = dest_t.astype(jnp.int32)


_hash_sort = pl.pallas_call(
    _hash_sort_body,
    grid=(B,),
    in_specs=[
        pl.BlockSpec((1, N, D), lambda b: (b, 0, 0)),
        pl.BlockSpec((D, NBINS // 2), lambda b: (0, 0)),
    ],
    out_specs=pl.BlockSpec((1, N, 1), lambda b: (b, 0, 0)),
    out_shape=jax.ShapeDtypeStruct((B, N, 1), jnp.int32),
)


def _knn_body(x_ref, drow_ref, dcol_ref, vals_ref, cols_ref, rowf_ref):
    b = pl.program_id(0)
    x = x_ref[0]
    drow = drow_ref[0]
    dcol = dcol_ref[0]

    iota_sub = lax.broadcasted_iota(jnp.int32, (N, BS), 0).astype(jnp.float32)

    sim_rows = []
    olane_rows = []
    for c in range(NBINS):
        svec_col = lax.broadcasted_iota(jnp.int32, (BS, 1), 0) + c * BS
        oh = (drow == svec_col).astype(jnp.float32)
        svec_row = lax.broadcasted_iota(jnp.int32, (1, BS), 1) + c * BS
        ohT = dcol == svec_row
        order_lane = jnp.sum(jnp.where(ohT, iota_sub, 0.0), axis=0,
                             keepdims=True)
        olane_rows.append(jnp.broadcast_to(order_lane, (BS, BS)))
        binned = jnp.dot(oh, x, preferred_element_type=jnp.float32)
        sim_rows.append(
            lax.dot_general(binned, binned, (((1,), (1,)), ((), ())),
                            preferred_element_type=jnp.float32))

    sim = jnp.concatenate(sim_rows, axis=0)
    olane = jnp.concatenate(olane_rows, axis=0)
    sm = 1.0 / (1.0 + jnp.exp(-sim))

    iota_l = lax.broadcasted_iota(jnp.int32, (N, BS), 1).astype(jnp.float32)
    keymat = iota_l * float(N) + olane

    smod = lax.broadcasted_iota(jnp.int32, (N, 1), 0)
    smod = smod - (smod // BS) * BS
    diag = iota_l == smod.astype(jnp.float32)
    order_col = jnp.sum(jnp.where(diag, olane, 0.0), axis=1, keepdims=True)

    big = float(N * BS * 2)
    work = sm
    vlist = []
    klist = []
    for _ in range(K):
        m = jnp.max(work, axis=1, keepdims=True)
        kmin = jnp.min(jnp.where(work == m, keymat, big), axis=1,
                       keepdims=True)
        vlist.append(m)
        klist.append(kmin)
        work = jnp.where(keymat == kmin, -1.0, work)

    keys = jnp.concatenate(klist, axis=1)
    gcols = keys - jnp.floor(keys * (1.0 / float(N))) * float(N)
    vals_ref[...] = jnp.concatenate(vlist, axis=1)
    cols_ref[...] = gcols.astype(jnp.int32)
    rowf_ref[...] = order_col.astype(jnp.int32) + b * N


_knn_in_specs = [
    pl.BlockSpec((1, N, D), lambda b: (b, 0, 0)),
    pl.BlockSpec((1, 1, N), lambda b: (b, 0, 0)),
    pl.BlockSpec((1, N, 1), lambda b: (b, 0, 0)),
]
_knn_out_specs = [
    pl.BlockSpec((N, K), lambda b: (b, 0)),
    pl.BlockSpec((N, K), lambda b: (b, 0)),
    pl.BlockSpec((N, 1), lambda b: (b, 0)),
]
_knn_out_shape = [
    jax.ShapeDtypeStruct((NROWS, K), jnp.float32),
    jax.ShapeDtypeStruct((NROWS, K), jnp.int32),
    jax.ShapeDtypeStruct((NROWS, 1), jnp.int32),
]

_knn = pl.pallas_call(
    _knn_body,
    grid=(B,),
    in_specs=_knn_in_specs,
    out_specs=_knn_out_specs,
    out_shape=_knn_out_shape,
)


@functools.cache
def _make_sc_scatter():
    mesh = plsc.VectorSubcoreMesh(core_axis_name="c", subcore_axis_name="s")

    @functools.partial(
        pl.kernel,
        mesh=mesh,
        compiler_params=pltpu.CompilerParams(needs_layout_passes=False),
        out_type=jax.ShapeDtypeStruct((NROWS, N), jnp.float32),
        scratch_types=[
            pltpu.VMEM((TILE, N), jnp.float32),
            pltpu.VMEM((ROWS_PER_W, K), jnp.float32),
            pltpu.VMEM((ROWS_PER_W, K), jnp.int32),
            pltpu.VMEM((ITERS, TILE), jnp.int32),
        ],
    )
    def _sc_scatter(vals_hbm, cols_hbm, rowf_hbm, zrow_hbm, out_hbm,
                    obuf, vbuf, cbuf, rbuf):
        wid = lax.axis_index("s") * NC + lax.axis_index("c")
        base = wid * ROWS_PER_W
        pltpu.sync_copy(zrow_hbm, obuf)
        pltpu.sync_copy(vals_hbm.at[pl.ds(base, ROWS_PER_W)], vbuf)
        pltpu.sync_copy(cols_hbm.at[pl.ds(base, ROWS_PER_W)], cbuf)
        pltpu.sync_copy(rowf_hbm.at[pl.ds(wid * ITERS, ITERS)], rbuf)
        zeros16 = jnp.zeros((K,), jnp.float32)
        for it in range(ITERS):
            for j in range(TILE):
                rsp = jnp.full((K,), j, jnp.int32)
                plsc.store_scatter(obuf, [rsp, cbuf[it * TILE + j, :]],
                                   vbuf[it * TILE + j, :])
            pltpu.sync_copy(obuf, out_hbm.at[rbuf.at[it]])
            for j in range(TILE):
                rsp = jnp.full((K,), j, jnp.int32)
                plsc.store_scatter(obuf, [rsp, cbuf[it * TILE + j, :]],
                                   zeros16)

    return _sc_scatter


def kernel(x, codebook):
    dest = _hash_sort(x, codebook)
    drow = dest.reshape(B, 1, N)
    vals, cols, rowf = _knn(x, drow, dest)
    rowf2 = rowf.reshape(NROWS // TILE, TILE)
    zrow = jnp.zeros((TILE, N), jnp.float32)
    a = _make_sc_scatter()(vals, cols, rowf2, zrow)
    return a.reshape(B, N, N)
